# baseline (speedup 1.0000x reference)
import functools

import jax
import jax.numpy as jnp
from jax import lax
from jax.experimental import pallas as pl
from jax.experimental.pallas import tpu as pltpu
from jax.experimental.pallas import tpu_sc as plsc

N_NODES = 10000
D_FEAT = 256
N_EDGES = 160000

NC = 2
NS = 16
DH = D_FEAT // NC
CHUNK = 128
NCHUNK = 80
HALF = NCHUNK // 2
EPAD = NS * NCHUNK * CHUNK
ROWS_PAD = 10240
STRIPE = ROWS_PAD // NS
DUMMY_ROW = N_NODES

_sc_mesh = plsc.VectorSubcoreMesh(core_axis_name="c", subcore_axis_name="s")


@functools.partial(
    pl.kernel,
    out_type=jax.ShapeDtypeStruct((NC, ROWS_PAD, DH), jnp.float32),
    mesh=_sc_mesh,
    scratch_types=[
        pltpu.VMEM((HALF, CHUNK), jnp.int32),
        pltpu.VMEM((HALF, CHUNK), jnp.int32),
        pltpu.VMEM((2, CHUNK, DH), jnp.float32),
        pltpu.VMEM_SHARED((ROWS_PAD, DH), jnp.float32),
        pltpu.SemaphoreType.DMA,
    ],
)
def _sc_agg(x_hbm, src_hbm, dst_hbm, out_hbm,
            src_v, dst_v, rows_v, agg_sh, sem):
    c = lax.axis_index("c")
    s = lax.axis_index("s")

    def _zrow(r, carry):
        for q in range(DH // 16):
            rows_v[0, r, pl.ds(q * 16, 16)] = jnp.zeros((16,), jnp.float32)
        return carry
    lax.fori_loop(0, CHUNK, _zrow, 0)

    def _zstripe(k, carry):
        pltpu.sync_copy(rows_v.at[0],
                        agg_sh.at[pl.ds(s * STRIPE + k * CHUNK, CHUNK)])
        return carry
    lax.fori_loop(0, STRIPE // CHUNK, _zstripe, 0)
    plsc.subcore_barrier()

    def _g(j, b):
        return pltpu.make_async_copy(
            x_hbm.at[:, pl.ds(c * DH, DH)].at[src_v.at[j]],
            rows_v.at[b], sem)

    for h in range(2):
        pltpu.sync_copy(src_hbm.at[s, pl.ds(h * HALF, HALF)], src_v)
        pltpu.sync_copy(dst_hbm.at[s, pl.ds(h * HALF, HALF)], dst_v)
        _g(0, 0).start()

        def _pair(p, carry):
            j0 = 2 * p
            _g(j0, 0).wait()
            _g(j0 + 1, 1).start()
            pltpu.sync_copy(rows_v.at[0], agg_sh.at[dst_v.at[j0]],
                            add=True)
            _g(j0 + 1, 1).wait()
            _g(j0 + 2, 0).start()
            pltpu.sync_copy(rows_v.at[1], agg_sh.at[dst_v.at[j0 + 1]],
                            add=True)
            return carry
        lax.fori_loop(0, HALF // 2 - 1, _pair, 0)
        _g(HALF - 2, 0).wait()
        _g(HALF - 1, 1).start()
        pltpu.sync_copy(rows_v.at[0], agg_sh.at[dst_v.at[HALF - 2]],
                        add=True)
        _g(HALF - 1, 1).wait()
        pltpu.sync_copy(rows_v.at[1], agg_sh.at[dst_v.at[HALF - 1]],
                        add=True)
    plsc.subcore_barrier()

    pltpu.sync_copy(agg_sh.at[pl.ds(s * STRIPE, STRIPE)],
                    out_hbm.at[c, pl.ds(s * STRIPE, STRIPE)])


RB = 5120
NBLK = ROWS_PAD // RB


def _tc_tail(a_ref, w1_ref, b1_ref, w2_ref, b2_ref, out_ref, acc_ref):
    i = pl.program_id(0)
    a = a_ref[...]
    w = w1_ref[...]
    z = (jnp.dot(a[0], w[0], preferred_element_type=jnp.float32)
         + jnp.dot(a[1], w[1], preferred_element_type=jnp.float32)
         + b1_ref[...])
    rows = i * RB + lax.broadcasted_iota(jnp.int32, (RB, 1), 0)
    h = jnp.where(rows < N_NODES, jnp.maximum(z, 0.0), 0.0)
    part = jnp.sum(h, axis=0, keepdims=True)

    @pl.when(i == 0)
    def _():
        acc_ref[...] = part

    @pl.when(i > 0)
    def _():
        acc_ref[...] = acc_ref[...] + part

    @pl.when(i == NBLK - 1)
    def _():
        out_ref[...] = (jnp.sum(acc_ref[...] * w2_ref[...], axis=1,
                                keepdims=True) + b2_ref[...])


_tc_call = pl.pallas_call(
    _tc_tail,
    grid=(NBLK,),
    in_specs=[
        pl.BlockSpec((NC, RB, DH), lambda i: (0, i, 0)),
        pl.BlockSpec((NC, DH, D_FEAT), lambda i: (0, 0, 0)),
        pl.BlockSpec((1, D_FEAT), lambda i: (0, 0)),
        pl.BlockSpec((1, D_FEAT), lambda i: (0, 0)),
        pl.BlockSpec((1, 1), lambda i: (0, 0)),
    ],
    out_specs=pl.BlockSpec((1, 1), lambda i: (0, 0)),
    out_shape=jax.ShapeDtypeStruct((1, 1), jnp.float32),
    scratch_shapes=[pltpu.VMEM((1, D_FEAT), jnp.float32)],
)


def kernel(x, edge_index, W1, b1, W2, b2):
    src = edge_index[0].astype(jnp.int32)
    dst = edge_index[1].astype(jnp.int32)
    pad = EPAD - N_EDGES
    src_p = jnp.concatenate([src, jnp.zeros((pad,), jnp.int32)])
    dummy = DUMMY_ROW + (jnp.arange(pad, dtype=jnp.int32)
                         % (ROWS_PAD - N_NODES))
    dst_p = jnp.concatenate([dst, dummy])
    src_r = src_p.reshape(NS, NCHUNK, CHUNK)
    dst_r = dst_p.reshape(NS, NCHUNK, CHUNK)
    agg2 = _sc_agg(x, src_r, dst_r)

    w1r = W1.reshape(NC, DH, D_FEAT)
    b1r = b1.reshape(1, D_FEAT)
    w2r = W2.reshape(1, D_FEAT)
    b2r = b2.reshape(1, 1)
    return _tc_call(agg2, w1r, b1r, w2r, b2r)


# speedup vs baseline: 1.2193x; 1.2193x over previous
import functools

import jax
import jax.numpy as jnp
from jax import lax
from jax.experimental import pallas as pl
from jax.experimental.pallas import tpu as pltpu
from jax.experimental.pallas import tpu_sc as plsc

N_NODES = 10000
D_FEAT = 256
N_EDGES = 160000

NC = 2
NS = 16
DH = D_FEAT // NC
CHUNK = 128
NCHUNK = 79
EPAD = NS * NCHUNK * CHUNK
ROWS_PAD = 10240
STRIPE = ROWS_PAD // NS
DUMMY_ROW = N_NODES

_sc_mesh = plsc.VectorSubcoreMesh(core_axis_name="c", subcore_axis_name="s")


@functools.partial(
    pl.kernel,
    out_type=jax.ShapeDtypeStruct((NC, ROWS_PAD, DH), jnp.float32),
    mesh=_sc_mesh,
    scratch_types=[
        pltpu.VMEM((NCHUNK, CHUNK), jnp.int32),
        pltpu.VMEM((NCHUNK, CHUNK), jnp.int32),
        pltpu.VMEM((CHUNK, DH), jnp.float32),
        pltpu.VMEM_SHARED((ROWS_PAD, DH), jnp.float32),
        pltpu.SemaphoreType.DMA,
    ],
)
def _sc_agg(x_hbm, src_hbm, dst_hbm, out_hbm,
            src_v, dst_v, rows_v, agg_sh, sem):
    c = lax.axis_index("c")
    s = lax.axis_index("s")

    def _zrow(r, carry):
        for q in range(DH // 16):
            rows_v[r, pl.ds(q * 16, 16)] = jnp.zeros((16,), jnp.float32)
        return carry
    lax.fori_loop(0, CHUNK, _zrow, 0)

    def _zstripe(k, carry):
        pltpu.sync_copy(rows_v,
                        agg_sh.at[pl.ds(s * STRIPE + k * CHUNK, CHUNK)])
        return carry
    lax.fori_loop(0, STRIPE // CHUNK, _zstripe, 0)
    plsc.subcore_barrier()

    pltpu.sync_copy(src_hbm.at[s], src_v)
    pltpu.sync_copy(dst_hbm.at[s], dst_v)

    def _step(j, carry):
        pltpu.async_copy(
            x_hbm.at[:, pl.ds(c * DH, DH)].at[src_v.at[j]],
            rows_v, sem).wait()
        pltpu.sync_copy(rows_v, agg_sh.at[dst_v.at[j]], add=True)
        return carry
    lax.fori_loop(0, NCHUNK, _step, 0)
    plsc.subcore_barrier()

    pltpu.sync_copy(agg_sh.at[pl.ds(s * STRIPE, STRIPE)],
                    out_hbm.at[c, pl.ds(s * STRIPE, STRIPE)])


RB = 5120
NBLK = ROWS_PAD // RB


def _tc_tail(a_ref, w1_ref, b1_ref, w2_ref, b2_ref, out_ref, acc_ref):
    i = pl.program_id(0)
    a = a_ref[...]
    w = w1_ref[...]
    z = (jnp.dot(a[0], w[0], preferred_element_type=jnp.float32)
         + jnp.dot(a[1], w[1], preferred_element_type=jnp.float32)
         + b1_ref[...])
    rows = i * RB + lax.broadcasted_iota(jnp.int32, (RB, 1), 0)
    h = jnp.where(rows < N_NODES, jnp.maximum(z, 0.0), 0.0)
    part = jnp.sum(h, axis=0, keepdims=True)

    @pl.when(i == 0)
    def _():
        acc_ref[...] = part

    @pl.when(i > 0)
    def _():
        acc_ref[...] = acc_ref[...] + part

    @pl.when(i == NBLK - 1)
    def _():
        out_ref[...] = (jnp.sum(acc_ref[...] * w2_ref[...], axis=1,
                                keepdims=True) + b2_ref[...])


_tc_call = pl.pallas_call(
    _tc_tail,
    grid=(NBLK,),
    in_specs=[
        pl.BlockSpec((NC, RB, DH), lambda i: (0, i, 0)),
        pl.BlockSpec((NC, DH, D_FEAT), lambda i: (0, 0, 0)),
        pl.BlockSpec((1, D_FEAT), lambda i: (0, 0)),
        pl.BlockSpec((1, D_FEAT), lambda i: (0, 0)),
        pl.BlockSpec((1, 1), lambda i: (0, 0)),
    ],
    out_specs=pl.BlockSpec((1, 1), lambda i: (0, 0)),
    out_shape=jax.ShapeDtypeStruct((1, 1), jnp.float32),
    scratch_shapes=[pltpu.VMEM((1, D_FEAT), jnp.float32)],
)


def kernel(x, edge_index, W1, b1, W2, b2):
    src = edge_index[0].astype(jnp.int32)
    dst = edge_index[1].astype(jnp.int32)
    pad = EPAD - N_EDGES
    src_p = jnp.concatenate([src, jnp.zeros((pad,), jnp.int32)])
    dummy = DUMMY_ROW + (jnp.arange(pad, dtype=jnp.int32)
                         % (ROWS_PAD - N_NODES))
    dst_p = jnp.concatenate([dst, dummy])
    src_r = src_p.reshape(NS, NCHUNK, CHUNK)
    dst_r = dst_p.reshape(NS, NCHUNK, CHUNK)
    agg2 = _sc_agg(x, src_r, dst_r)

    w1r = W1.reshape(NC, DH, D_FEAT)
    b1r = b1.reshape(1, D_FEAT)
    w2r = W2.reshape(1, D_FEAT)
    b2r = b2.reshape(1, 1)
    return _tc_call(agg2, w1r, b1r, w2r, b2r)
